# baseline (speedup 1.0000x reference)
import functools

import jax
import jax.numpy as jnp
from jax import lax
from jax.experimental import pallas as pl
from jax.experimental.pallas import tpu as pltpu
from jax.experimental.pallas import tpu_sc as plsc

NUM_EXPERTS = 64
TOP_K = 8
D_MODEL = 4096
TOKENS = 16384

T_BLK = 1024

N_WORKERS = 32
TPW = TOKENS // N_WORKERS


def _router_block(x_ref, w_ref, b_ref, p_out, lbl_out, zl_out,
                  psum_acc, zsum_acc):
    step = pl.program_id(0)
    nsteps = pl.num_programs(0)

    logits = jax.lax.dot_general(
        x_ref[...], w_ref[...],
        dimension_numbers=(((1,), (1,)), ((), ())),
        preferred_element_type=jnp.float32,
    )

    lt = logits.T + b_ref[...]
    zpart = jnp.sum(lt * lt, axis=1, keepdims=True)

    m = jnp.max(lt, axis=0, keepdims=True)
    e = jnp.exp(lt - m)
    s = jnp.sum(e, axis=0, keepdims=True)
    probs = e / s

    ppart = jnp.sum(probs, axis=1, keepdims=True)

    @pl.when(step == 0)
    def _init():
        psum_acc[...] = ppart
        zsum_acc[...] = zpart

    @pl.when(step != 0)
    def _accum():
        psum_acc[...] += ppart
        zsum_acc[...] += zpart

    @pl.when(step == nsteps - 1)
    def _finalize():
        tpe = psum_acc[...] / TOKENS
        u = 1.0 / NUM_EXPERTS
        lbl_out[0, 0] = jnp.sum((tpe - u) ** 2) * NUM_EXPERTS
        zl_out[0, 0] = jnp.sum(zsum_acc[...]) / (TOKENS * NUM_EXPERTS) * 0.001

    p_out[...] = probs.T


def _tc_probs(x, W, expert_bias):
    grid = TOKENS // T_BLK
    bias = expert_bias.reshape(NUM_EXPERTS, 1)
    return pl.pallas_call(
        _router_block,
        grid=(grid,),
        in_specs=[
            pl.BlockSpec((T_BLK, D_MODEL), lambda i: (i, 0)),
            pl.BlockSpec((NUM_EXPERTS, D_MODEL), lambda i: (0, 0)),
            pl.BlockSpec((NUM_EXPERTS, 1), lambda i: (0, 0)),
        ],
        out_specs=[
            pl.BlockSpec((T_BLK, NUM_EXPERTS), lambda i: (i, 0)),
            pl.BlockSpec(memory_space=pltpu.SMEM),
            pl.BlockSpec(memory_space=pltpu.SMEM),
        ],
        out_shape=[
            jax.ShapeDtypeStruct((TOKENS, NUM_EXPERTS), jnp.float32),
            jax.ShapeDtypeStruct((1, 1), jnp.float32),
            jax.ShapeDtypeStruct((1, 1), jnp.float32),
        ],
        scratch_shapes=[
            pltpu.VMEM((NUM_EXPERTS, 1), jnp.float32),
            pltpu.VMEM((NUM_EXPERTS, 1), jnp.float32),
        ],
        compiler_params=pltpu.CompilerParams(
            dimension_semantics=("arbitrary",),
        ),
    )(x, W, bias)


def _sc_topk_body(probs_hbm, w_hbm, i_hbm, probs_v, w_v, i_v):
    wid = lax.axis_index("s") * 2 + lax.axis_index("c")
    base = wid * TPW
    pltpu.sync_copy(probs_hbm.at[pl.ds(base, TPW)], probs_v)

    lane = lax.iota(jnp.int32, 16)
    low8 = lane < 8

    def merge(ak, av, bk, bv):
        mk = jnp.where(low8, ak, lax.rev(bk, (0,)))
        mv = jnp.where(low8, av, lax.rev(bv, (0,)))
        return plsc.sort_key_val(mk, mv, descending=True)

    def body(t, carry):
        ks = []
        vs = []
        for c in range(NUM_EXPERTS // 16):
            kc = probs_v[t, pl.ds(c * 16, 16)]
            ic = lane + c * 16
            sk, sv = plsc.sort_key_val(kc, ic, descending=True)
            ks.append(sk)
            vs.append(sv)
        k01, v01 = merge(ks[0], vs[0], ks[1], vs[1])
        k23, v23 = merge(ks[2], vs[2], ks[3], vs[3])
        fk, fv = merge(k01, v01, k23, v23)

        wsum = jnp.sum(jnp.where(low8, fk, 0.0))
        wn = fk / (wsum + 1e-8)
        plsc.store_compressed(w_v.at[pl.ds(t * TOP_K, 16)], wn, mask=low8)
        plsc.store_compressed(i_v.at[pl.ds(t * TOP_K, 16)], fv, mask=low8)
        return carry

    lax.fori_loop(0, TPW, body, 0)

    pltpu.sync_copy(w_v.at[pl.ds(0, TPW * TOP_K)],
                    w_hbm.at[pl.ds(base * TOP_K, TPW * TOP_K)])
    pltpu.sync_copy(i_v.at[pl.ds(0, TPW * TOP_K)],
                    i_hbm.at[pl.ds(base * TOP_K, TPW * TOP_K)])


def _sc_topk(probs):
    mesh = plsc.VectorSubcoreMesh(core_axis_name="c", subcore_axis_name="s")
    run = functools.partial(
        pl.kernel,
        mesh=mesh,
        out_type=[
            jax.ShapeDtypeStruct((TOKENS * TOP_K,), jnp.float32),
            jax.ShapeDtypeStruct((TOKENS * TOP_K,), jnp.int32),
        ],
        scratch_types=[
            pltpu.VMEM((TPW, NUM_EXPERTS), jnp.float32),
            pltpu.VMEM((TPW * TOP_K + 8,), jnp.float32),
            pltpu.VMEM((TPW * TOP_K + 8,), jnp.int32),
        ],
        compiler_params=pltpu.CompilerParams(needs_layout_passes=False),
    )(_sc_topk_body)
    w_flat, i_flat = run(probs)
    return (w_flat.reshape(TOKENS, TOP_K), i_flat.reshape(TOKENS, TOP_K))


@jax.jit
def kernel(x, W, expert_bias):
    probs, lbl, zl = _tc_probs(x, W, expert_bias)
    w_out, i_out = _sc_topk(probs)
    return (w_out, i_out, lbl.reshape(()), zl.reshape(()))


# speedup vs baseline: 1.5949x; 1.5949x over previous
import jax
import jax.numpy as jnp
from jax.experimental import pallas as pl
from jax.experimental.pallas import tpu as pltpu

NUM_EXPERTS = 64
TOP_K = 8
D_MODEL = 4096
TOKENS = 16384

T_BLK = 1024


def _router_block(x_ref, w_ref, b_ref, w_out, i_out, lbl_out, zl_out,
                  psum_acc, zsum_acc):
    step = pl.program_id(0)
    nsteps = pl.num_programs(0)

    logits = jax.lax.dot_general(
        x_ref[...], w_ref[...],
        dimension_numbers=(((1,), (1,)), ((), ())),
        preferred_element_type=jnp.float32,
    )

    lt = logits.T + b_ref[...]

    zpart = jnp.sum(lt * lt, axis=1, keepdims=True)

    m = jnp.max(lt, axis=0, keepdims=True)
    e = jnp.exp(lt - m)
    s = jnp.sum(e, axis=0, keepdims=True)
    probs = e / s

    ppart = jnp.sum(probs, axis=1, keepdims=True)

    @pl.when(step == 0)
    def _init():
        psum_acc[...] = ppart
        zsum_acc[...] = zpart

    @pl.when(step != 0)
    def _accum():
        psum_acc[...] += ppart
        zsum_acc[...] += zpart

    @pl.when(step == nsteps - 1)
    def _finalize():
        tpe = psum_acc[...] / TOKENS
        u = 1.0 / NUM_EXPERTS
        lbl_out[0, 0] = jnp.sum((tpe - u) ** 2) * NUM_EXPERTS
        zl_out[0, 0] = jnp.sum(zsum_acc[...]) / (TOKENS * NUM_EXPERTS) * 0.001

    sub = jax.lax.broadcasted_iota(jnp.int32, probs.shape, 0)
    vals = probs
    ws = []
    idxs = []
    for _ in range(TOP_K):
        mk = jnp.max(vals, axis=0, keepdims=True)
        is_mk = vals >= mk
        idx = jnp.min(
            jnp.where(is_mk, sub, NUM_EXPERTS), axis=0, keepdims=True
        )
        ws.append(mk)
        idxs.append(idx)
        vals = jnp.where(sub == idx, -1.0, vals)

    w_cat = jnp.concatenate(ws, axis=0)
    wsum = jnp.sum(w_cat, axis=0, keepdims=True)
    w_out[...] = (w_cat / (wsum + 1e-8)).T
    i_out[...] = jnp.concatenate(idxs, axis=0).T


@jax.jit
def kernel(x, W, expert_bias):
    grid = TOKENS // T_BLK
    bias = expert_bias.reshape(NUM_EXPERTS, 1)

    w_out, i_out, lbl, zl = pl.pallas_call(
        _router_block,
        grid=(grid,),
        in_specs=[
            pl.BlockSpec((T_BLK, D_MODEL), lambda i: (i, 0)),
            pl.BlockSpec((NUM_EXPERTS, D_MODEL), lambda i: (0, 0)),
            pl.BlockSpec((NUM_EXPERTS, 1), lambda i: (0, 0)),
        ],
        out_specs=[
            pl.BlockSpec((T_BLK, TOP_K), lambda i: (i, 0)),
            pl.BlockSpec((T_BLK, TOP_K), lambda i: (i, 0)),
            pl.BlockSpec(memory_space=pltpu.SMEM),
            pl.BlockSpec(memory_space=pltpu.SMEM),
        ],
        out_shape=[
            jax.ShapeDtypeStruct((TOKENS, TOP_K), jnp.float32),
            jax.ShapeDtypeStruct((TOKENS, TOP_K), jnp.int32),
            jax.ShapeDtypeStruct((1, 1), jnp.float32),
            jax.ShapeDtypeStruct((1, 1), jnp.float32),
        ],
        scratch_shapes=[
            pltpu.VMEM((NUM_EXPERTS, 1), jnp.float32),
            pltpu.VMEM((NUM_EXPERTS, 1), jnp.float32),
        ],
        compiler_params=pltpu.CompilerParams(
            dimension_semantics=("arbitrary",),
        ),
    )(x, W, bias)

    return (w_out, i_out, lbl.reshape(()), zl.reshape(()))
